# Initial kernel scaffold; baseline (speedup 1.0000x reference)
import jax
import jax.numpy as jnp
from jax.experimental import pallas as pl


def kernel(x, batch, params):
    raise NotImplementedError("write your pallas kernel here")


# speedup vs baseline: 11.8934x; 11.8934x over previous
import functools

import jax
import jax.numpy as jnp
import numpy as np
from jax import lax
from jax.experimental import pallas as pl
from jax.experimental.pallas import tpu as pltpu
from jax.experimental.pallas import tpu_sc as plsc

F32 = jnp.float32
I32 = jnp.int32

NUM_GRAPHS = 20
NGP = 32
SENTINEL = 99.0
R = 512
C = 512
INF = np.float32(3.0e38)

_INTERPRET = False


def _pc(body, out_shape, **kw):
    return pl.pallas_call(body, out_shape=out_shape, interpret=_INTERPRET, **kw)


def _elu(x):
    xn = jnp.minimum(x, 0.0)
    taylor = xn * (1.0 + xn * (0.5 + xn * (1.0 / 6.0 + xn * (1.0 / 24.0))))
    em1 = jnp.where(xn > -0.05, taylor, jnp.exp(xn) - 1.0)
    return jnp.where(x > 0, x, em1)


def _masked_stats(x, maskf):
    xm = x * maskf
    s = jnp.sum(xm, axis=0, keepdims=True)
    s2 = jnp.sum(xm * x, axis=0, keepdims=True)
    return jnp.concatenate([s, s2], axis=0)


def _bn_apply(x, stats, g, b, n_real, maskf):
    mean = stats[0:1, :] / n_real
    diff = x - mean
    var = jnp.sum(diff * diff * maskf, axis=0, keepdims=True) / n_real
    return diff * lax.rsqrt(var + 1e-5) * g + b


BF16 = jnp.bfloat16


def _dotb(a, b):
    return jnp.dot(a.astype(BF16), b.astype(BF16),
                   preferred_element_type=F32)


def _onehot(batchf):
    ids = lax.broadcasted_iota(I32, (batchf.shape[0], NGP), 1).astype(F32)
    return jnp.where(batchf == ids, 1.0, 0.0)


def _k_stats(x, batchf, n_real):
    def body(x_ref, b_ref, o_ref):
        maskf = jnp.where(b_ref[...] < NUM_GRAPHS, 1.0, 0.0)
        st = _masked_stats(x_ref[...], maskf)
        o_ref[...] = jnp.concatenate(
            [st, jnp.zeros((6, st.shape[1]), F32)], axis=0)
    return _pc(body, jax.ShapeDtypeStruct((8, x.shape[1]), F32))(x, batchf)


def _k_bn0_seg(x, batchf, stats, g, b, n_real):
    def body(x_ref, bf_ref, st_ref, g_ref, b_ref, xn_ref, seg_ref, cnt_ref):
        maskf = jnp.where(bf_ref[...] < NUM_GRAPHS, 1.0, 0.0)
        xn = _bn_apply(x_ref[...], st_ref[...], g_ref[...], b_ref[...], n_real,
                       maskf)
        xn_ref[...] = xn
        oh = _onehot(bf_ref[...])
        seg_ref[...] = lax.dot_general(oh, xn, (((0,), (0,)), ((), ())),
                                       preferred_element_type=F32,
                                       precision=lax.Precision.HIGHEST)
        cnt_ref[...] = lax.dot_general(oh, jnp.ones((oh.shape[0], 1), F32),
                                       (((0,), (0,)), ((), ())),
                                       preferred_element_type=F32,
                                       precision=lax.Precision.HIGHEST)
    return _pc(body, (
        jax.ShapeDtypeStruct(x.shape, F32),
        jax.ShapeDtypeStruct((NGP, x.shape[1]), F32),
        jax.ShapeDtypeStruct((NGP, 1), F32),
    ))(x, batchf, stats, g, b)


def _k_blockA(xprev, batchf, seg, cnt, W0, b0, W1, b1):
    def body(x_ref, bf_ref, seg_ref, cnt_ref, W0_ref, b0_ref, W1_ref, b1_ref,
             h2_ref, st_ref):
        bf = bf_ref[...]
        oh = _onehot(bf)
        segmean = seg_ref[...] / jnp.maximum(cnt_ref[...], 1.0)
        pooled = jnp.dot(oh, segmean, preferred_element_type=F32,
                         precision=lax.Precision.HIGHEST)
        nf = x_ref.shape[1]
        h1 = _elu(_dotb(x_ref[...], W0_ref[0:nf, :])
                  + _dotb(pooled, W0_ref[nf:, :])
                  + b0_ref[...])
        h2 = _elu(_dotb(h1, W1_ref[...]) + b1_ref[...])
        h2_ref[...] = h2
        maskf = jnp.where(bf < NUM_GRAPHS, 1.0, 0.0)
        st = _masked_stats(h2, maskf)
        st_ref[...] = jnp.concatenate(
            [st, jnp.zeros((6, st.shape[1]), F32)], axis=0)
    return _pc(body, (
        jax.ShapeDtypeStruct((xprev.shape[0], W1.shape[1]), F32),
        jax.ShapeDtypeStruct((8, W1.shape[1]), F32),
    ))(xprev, batchf, seg, cnt, W0, b0, W1, b1)


def _k_blockB(h2, batchf, stats, g, b, W2, b2, Ws, bs, Wh, bh, n_real):
    def body(h2_ref, bf_ref, st_ref, g_ref, b_ref, W2_ref, b2_ref, Ws_ref,
             bs_ref, Wh_ref, bh_ref, xg_ref, h_ref, s8_ref):
        maskf = jnp.where(bf_ref[...] < NUM_GRAPHS, 1.0, 0.0)
        z = _bn_apply(h2_ref[...], st_ref[...], g_ref[...], b_ref[...], n_real,
                      maskf)
        xg = _elu(_dotb(z, W2_ref[...]) + b2_ref[...])
        xg_ref[...] = xg
        h_ref[...] = _dotb(xg, Wh_ref[...]) + bh_ref[...]
        s = _dotb(xg, Ws_ref[...]) + bs_ref[...]
        sq = jnp.sum(s * s, axis=1, keepdims=True)
        s8_ref[...] = jnp.concatenate(
            [s, sq, jnp.zeros((s.shape[0], 3), F32)], axis=1)
    n = h2.shape[0]
    return _pc(body, (
        jax.ShapeDtypeStruct((n, W2.shape[1]), F32),
        jax.ShapeDtypeStruct((n, Wh.shape[1]), F32),
        jax.ShapeDtypeStruct((n, 8), F32),
    ))(h2, batchf, stats, g, b, W2, b2, Ws, bs, Wh, bh)


def _k_knn(s8, batchf, batchrow):
    NP = s8.shape[0]
    nrb, ncb = NP // R, NP // C

    def body(s8r_ref, sT_ref, br_ref, bc2_ref, idx_ref, w_ref,
             tile_ref, tilex_ref, runv_ref, runi_ref, runx_ref):
        j = pl.program_id(1)

        @pl.when(j == 0)
        def _init():
            runv_ref[...] = jnp.full((R, 16), INF, F32)
            runi_ref[...] = jnp.zeros((R, 16), F32)
            runx_ref[...] = jnp.full((R, 16), INF, F32)

        br = br_ref[...]
        bc2 = bc2_ref[...]
        active = jnp.logical_and(bc2[0, C - 1] >= br[0, 0],
                                 br[R - 1, 0] >= bc2[0, 0])

        @pl.when(active)
        def _compute():
            s8r = s8r_ref[...]
            sT = sT_ref[...]
            mask = br == bc2
            cross = jnp.dot(s8r[:, 0:4].astype(BF16),
                            sT[0:4, :].astype(BF16),
                            preferred_element_type=F32)
            dsel = (s8r[:, 4:5] + sT[4:5, :]) - 2.0 * cross
            tile_ref[...] = jnp.where(mask, dsel, INF)
            d0 = s8r[:, 0:1] - sT[0:1, :]
            d1 = s8r[:, 1:2] - sT[1:2, :]
            d2_ = s8r[:, 2:3] - sT[2:3, :]
            d3 = s8r[:, 3:4] - sT[3:4, :]
            dex = ((d0 * d0 + d1 * d1) + d2_ * d2_) + d3 * d3
            tilex_ref[...] = jnp.where(mask, dex, INF)
            iota_c = lax.broadcasted_iota(I32, (R, C), 1).astype(F32)
            iota16 = lax.broadcasted_iota(I32, (R, 16), 1).astype(F32)
            colbase = (j * C).astype(F32)

            def wbody(carry):
                k, _ = carry
                t = tile_ref[...]
                m = jnp.min(t, axis=1, keepdims=True)
                rv = runv_ref[...]
                rmax = jnp.max(rv, axis=1, keepdims=True)
                repl = m < rmax
                pos = jnp.min(jnp.where(t == m, iota_c, np.float32(1e9)),
                              axis=1, keepdims=True)
                gidx = colbase + pos
                rpos = jnp.min(jnp.where(rv == rmax, iota16, np.float32(99.0)),
                               axis=1, keepdims=True)
                oh = jnp.logical_and(iota16 == rpos, repl)
                sel = iota_c == pos
                mx = jnp.min(jnp.where(sel, tilex_ref[...], INF),
                             axis=1, keepdims=True)
                runv_ref[...] = jnp.where(oh, m, rv)
                runi_ref[...] = jnp.where(oh, gidx, runi_ref[...])
                runx_ref[...] = jnp.where(oh, mx, runx_ref[...])
                tile_ref[...] = jnp.where(sel, INF, t)
                return k + 1, jnp.any(repl)

            def wcond(carry):
                k, cont = carry
                return jnp.logical_and(k < 16, cont)

            lax.while_loop(wcond, wbody, (np.int32(0), True))

        @pl.when(j == ncb - 1)
        def _finalize():
            rx = runx_ref[...]
            w_ref[...] = jnp.exp(-10.0 * jnp.minimum(rx, np.float32(1e4)))
            idx_ref[...] = runi_ref[...].astype(I32)

    return _pc(
        body,
        (jax.ShapeDtypeStruct((NP, 16), I32),
         jax.ShapeDtypeStruct((NP, 16), F32)),
        grid=(nrb, ncb),
        in_specs=[
            pl.BlockSpec((R, 8), lambda i, j: (i, 0)),
            pl.BlockSpec((8, C), lambda i, j: (0, j)),
            pl.BlockSpec((R, 1), lambda i, j: (i, 0)),
            pl.BlockSpec((1, C), lambda i, j: (0, j)),
        ],
        out_specs=[
            pl.BlockSpec((R, 16), lambda i, j: (i, 0)),
            pl.BlockSpec((R, 16), lambda i, j: (i, 0)),
        ],
        scratch_shapes=[
            pltpu.VMEM((R, C), F32),
            pltpu.VMEM((R, C), F32),
            pltpu.VMEM((R, 16), F32),
            pltpu.VMEM((R, 16), F32),
            pltpu.VMEM((R, 16), F32),
        ],
    )(s8, jnp.transpose(s8), batchrow, jnp.reshape(batchrow, (1, -1)))


def _k_aggregate(h, idx, w):
    NP = h.shape[0]
    D = h.shape[1]
    NW = 32
    per_w = NP // NW
    CH = 8
    nch = per_w // CH
    idxf = jnp.reshape(idx, (NP * 16,))
    wf = jnp.reshape(w, (NP * 16,))
    h128 = jnp.pad(h, ((0, 0), (0, 128 - D)))

    mesh = plsc.VectorSubcoreMesh(core_axis_name="c", subcore_axis_name="s")

    @functools.partial(
        pl.kernel,
        out_type=jax.ShapeDtypeStruct((NP, 2 * D), F32),
        mesh=mesh,
        scratch_types=[
            pltpu.VMEM((CH * 16,), I32),
            pltpu.VMEM((CH * 16, 128), F32),
            pltpu.VMEM((CH * 16,), F32),
            pltpu.VMEM((CH, 2 * D), F32),
            pltpu.SemaphoreType.DMA,
        ],
    )
    def agg_kernel(h_hbm, idx_hbm, w_hbm, out_hbm,
                   idx_v, rows_v, w_v, out_v, sem):
        cid = lax.axis_index("c")
        sid = lax.axis_index("s")
        wid = sid * 2 + cid
        node0 = wid * per_w

        def chunk(g, carry):
            base = node0 + g * CH
            pltpu.sync_copy(idx_hbm.at[pl.ds(base * 16, CH * 16)], idx_v)
            pltpu.sync_copy(w_hbm.at[pl.ds(base * 16, CH * 16)], w_v)
            pltpu.async_copy(h_hbm.at[idx_v], rows_v, sem).wait()
            for t in range(CH):
                accm = [jnp.zeros((16,), F32) for _ in range(D // 16)]
                accx = [jnp.full((16,), -INF, F32) for _ in range(D // 16)]
                wrow = w_v[pl.ds(t * 16, 16)]
                for r in range(16):
                    wbc = lax.gather(
                        wrow, jnp.full((16, 1), r, I32),
                        lax.GatherDimensionNumbers(
                            offset_dims=(), collapsed_slice_dims=(0,),
                            start_index_map=(0,)),
                        slice_sizes=(1,),
                        mode=lax.GatherScatterMode.PROMISE_IN_BOUNDS)
                    for fc in range(D // 16):
                        hv = rows_v[t * 16 + r, pl.ds(fc * 16, 16)]
                        v = wbc * hv
                        accm[fc] = accm[fc] + v
                        accx[fc] = jnp.maximum(accx[fc], v)
                for fc in range(D // 16):
                    out_v[t, pl.ds(fc * 16, 16)] = accm[fc] * (1.0 / 16.0)
                    out_v[t, pl.ds(D + fc * 16, 16)] = accx[fc]
            pltpu.sync_copy(out_v, out_hbm.at[pl.ds(base, CH), :])
            return carry

        lax.fori_loop(0, nch, chunk, 0)

    return agg_kernel(h128, idxf, wf)


def _k_blockE(xg, agg, batchf, W1o, W2o, b2o):
    def body(xg_ref, agg_ref, bf_ref, W1_ref, W2_ref, b2_ref, y_ref, st_ref):
        y = jnp.dot(xg_ref[...], W1_ref[...], preferred_element_type=F32) \
            + (jnp.dot(agg_ref[...], W2_ref[...], preferred_element_type=F32)
               + b2_ref[...])
        y_ref[...] = y
        maskf = jnp.where(bf_ref[...] < NUM_GRAPHS, 1.0, 0.0)
        st = _masked_stats(y, maskf)
        st_ref[...] = jnp.concatenate(
            [st, jnp.zeros((6, st.shape[1]), F32)], axis=0)
    n = xg.shape[0]
    return _pc(body, (
        jax.ShapeDtypeStruct((n, W2o.shape[1]), F32),
        jax.ShapeDtypeStruct((8, W2o.shape[1]), F32),
    ))(xg, agg, batchf, W1o, W2o, b2o)


def _k_blockF(y, stats, g, b, W3, b3, batchf, n_real, last):
    def body(y_ref, st_ref, g_ref, b_ref, W3_ref, b3_ref, bf_ref,
             xn_ref, ft_ref, seg_ref):
        maskf = jnp.where(bf_ref[...] < NUM_GRAPHS, 1.0, 0.0)
        xn = _bn_apply(y_ref[...], st_ref[...], g_ref[...], b_ref[...], n_real,
                       maskf)
        xn_ref[...] = xn
        ft_ref[...] = _elu(jnp.dot(xn, W3_ref[...], preferred_element_type=F32)
                           + b3_ref[...])
        oh = _onehot(bf_ref[...])
        seg_ref[...] = lax.dot_general(oh, xn, (((0,), (0,)), ((), ())),
                                      preferred_element_type=F32,
                                      precision=lax.Precision.HIGHEST)
    n = y.shape[0]
    return _pc(body, (
        jax.ShapeDtypeStruct((n, y.shape[1]), F32),
        jax.ShapeDtypeStruct((n, W3.shape[1]), F32),
        jax.ShapeDtypeStruct((NGP, y.shape[1]), F32),
    ))(y, stats, g, b, W3, b3, batchf)


def _k_head(feats, Wd, bd, Wh, bh):
    def body(f0_ref, f1_ref, f2_ref, f3_ref, Wd_ref, bd_ref, Wh_ref, bh_ref,
             o_ref):
        xcat = jnp.concatenate(
            [f0_ref[...], f1_ref[...], f2_ref[...], f3_ref[...]], axis=1)
        y = _elu(jnp.dot(xcat, Wd_ref[...], preferred_element_type=F32)
                 + bd_ref[...])
        o = jnp.dot(y, Wh_ref[...], preferred_element_type=F32) + bh_ref[...]
        sig = jax.nn.sigmoid(o)
        lanes = lax.broadcasted_iota(I32, o.shape, 1)
        use_sig = jnp.logical_or(lanes == 0, lanes == 9)
        o_ref[...] = jnp.where(use_sig, sig, o)
    n = feats[0].shape[0]
    return _pc(body, jax.ShapeDtypeStruct((n, Wh.shape[1]), F32))(
        *feats, Wd, bd, Wh, bh)


def kernel(x, batch, params):
    n_real = x.shape[0]
    NP = ((n_real + R - 1) // R) * R
    pad = NP - n_real

    xp = jnp.pad(x, ((0, pad), (0, 0)))
    batchf = jnp.pad(batch.astype(F32), (0, pad),
                     constant_values=SENTINEL)[:, None]

    def row(v):
        return jnp.reshape(v, (1, -1))

    st0 = _k_stats(xp, batchf, n_real)
    xn, seg, cnt = _k_bn0_seg(xp, batchf, st0,
                              row(params["bn0"]["g"]), row(params["bn0"]["b"]),
                              n_real)

    xcur = xn
    feats = []
    for bi, blk in enumerate(params["blocks"]):
        h2, st1 = _k_blockA(xcur, batchf, seg, cnt,
                            blk["lin0"]["W"], row(blk["lin0"]["b"]),
                            blk["lin1"]["W"], row(blk["lin1"]["b"]))
        xg, h, s8 = _k_blockB(h2, batchf, st1,
                              row(blk["bn1"]["g"]), row(blk["bn1"]["b"]),
                              blk["lin2"]["W"], row(blk["lin2"]["b"]),
                              blk["lin_s"]["W"], row(blk["lin_s"]["b"]),
                              blk["lin_h"]["W"], row(blk["lin_h"]["b"]),
                              n_real)
        idx, w = _k_knn(s8, batchf, batchf)
        agg = _k_aggregate(h, idx, w)
        y, st2 = _k_blockE(xg, agg, batchf,
                           blk["lin_out1"]["W"], blk["lin_out2"]["W"],
                           row(blk["lin_out2"]["b"]))
        xcur, ft, seg = _k_blockF(y, st2,
                                  row(blk["bn2"]["g"]),
                                  row(blk["bn2"]["b"]),
                                  blk["lin3"]["W"],
                                  row(blk["lin3"]["b"]),
                                  batchf, n_real, bi == 3)
        feats.append(ft)

    Wh = jnp.concatenate([params["p_beta"]["W"], params["p_ccoords"]["W"],
                          params["p_p"]["W"], params["p_vertex"]["W"],
                          params["p_charge"]["W"]], axis=1)
    bh = jnp.concatenate([params["p_beta"]["b"], params["p_ccoords"]["b"],
                          params["p_p"]["b"], params["p_vertex"]["b"],
                          params["p_charge"]["b"]])
    out = _k_head(feats, params["dense_cat"]["W"], row(params["dense_cat"]["b"]),
                  Wh, row(bh))
    return out[:n_real]
